# baseline (speedup 1.0000x reference)
import functools

import jax
import jax.numpy as jnp
from jax import lax
from jax.experimental import pallas as pl
from jax.experimental.pallas import tpu as pltpu
from jax.experimental.pallas import tpu_sc as plsc

N_NODES = 10000
N_EDGES = 320000
NS = 16
L = 16
CH = 128
VPC = CH // L
N_PAD = 10240
NODES_PT = N_PAD // NS
LAST_N = N_NODES - (NS - 1) * NODES_PT
CPT = -(-N_EDGES // (NS * CH))
EPT = CPT * CH
EPE = N_EDGES // NS
E_PAD = EPT * NS

_mesh = plsc.VectorSubcoreMesh(
    core_axis_name="c", subcore_axis_name="s", num_cores=1, num_subcores=NS)


@functools.partial(
    pl.kernel,
    out_type=jax.ShapeDtypeStruct((N_NODES,), jnp.float32),
    mesh=_mesh,
    compiler_params=pltpu.CompilerParams(needs_layout_passes=False),
    scratch_types=[
        pltpu.VMEM((N_PAD,), jnp.float32),
        pltpu.VMEM((EPT,), jnp.int32),
        pltpu.VMEM((CPT, CH), jnp.int32),
        pltpu.VMEM((EPT,), jnp.float32),
        pltpu.VMEM((CPT, CH), jnp.float32),
        pltpu.VMEM((CH,), jnp.float32),
        pltpu.VMEM((NODES_PT,), jnp.float32),
        pltpu.VMEM((NODES_PT,), jnp.float32),
        pltpu.VMEM((NODES_PT,), jnp.float32),
        pltpu.VMEM_SHARED((N_PAD,), jnp.float32),
        pltpu.VMEM_SHARED((N_PAD,), jnp.float32),
        pltpu.SemaphoreType.DMA,
        pltpu.SemaphoreType.DMA,
    ],
)
def _sc_kernel(xcol_hbm, ei_hbm, ea_hbm, dst_hbm, out_hbm,
               xcol_v, src_v, dst_v, ea_v, ld_v, ones_v,
               sv, cv, ov, sums_sh, cnts_sh, stage_sem, scat_sem):
    t = lax.axis_index("s")
    nbase = t * NODES_PT
    ebase = t * EPE

    c0 = pltpu.async_copy(xcol_hbm, xcol_v, stage_sem)
    c1 = pltpu.async_copy(ei_hbm.at[pl.ds(ebase, EPE)],
                          src_v.at[pl.ds(0, EPE)], stage_sem)
    c2 = pltpu.async_copy(dst_hbm.at[t], dst_v, stage_sem)
    c3 = pltpu.async_copy(ea_hbm.at[pl.ds(ebase, EPE)],
                          ea_v.at[pl.ds(0, EPE)], stage_sem)

    for i in range((EPT - EPE) // L):
        src_v[pl.ds(EPE + i * L, L)] = jnp.zeros((L,), jnp.int32)
        ea_v[pl.ds(EPE + i * L, L)] = jnp.ones((L,), jnp.float32)

    def zbody(i, _):
        ov[pl.ds(i * L, L)] = jnp.zeros((L,), jnp.float32)
        return 0
    lax.fori_loop(0, NODES_PT // L, zbody, 0)
    for i in range(VPC):
        ones_v[pl.ds(i * L, L)] = jnp.ones((L,), jnp.float32)
    pltpu.sync_copy(ov, sums_sh.at[pl.ds(nbase, NODES_PT)])
    pltpu.sync_copy(ov, cnts_sh.at[pl.ds(nbase, NODES_PT)])
    c0.wait()
    c1.wait()
    c2.wait()
    c3.wait()
    plsc.subcore_barrier()

    @plsc.parallel_loop(0, CPT)
    def _compute(j):
        for k in range(VPC):
            o = j * CH + k * L
            sl = pl.ds(k * L, L)
            xs = plsc.load_gather(xcol_v, [src_v[pl.ds(o, L)]])
            xd = plsc.load_gather(xcol_v, [dst_v[j, sl]])
            ld_v[j, sl] = (xd - xs) / ea_v[pl.ds(o, L)]

    def sbody(j, _):
        idx = dst_v.at[j]
        pltpu.async_copy(ld_v.at[j], sums_sh.at[idx], scat_sem, add=True)
        pltpu.async_copy(ones_v, cnts_sh.at[idx], scat_sem, add=True)

        @pl.when(j > 0)
        def _():
            pidx = dst_v.at[j - 1]
            pltpu.make_async_copy(ld_v.at[j - 1], sums_sh.at[pidx],
                                  scat_sem).wait()
            pltpu.make_async_copy(ones_v, cnts_sh.at[pidx], scat_sem).wait()
        return 0
    lax.fori_loop(0, CPT, sbody, 0)
    lidx = dst_v.at[CPT - 1]
    pltpu.make_async_copy(ld_v.at[CPT - 1], sums_sh.at[lidx], scat_sem).wait()
    pltpu.make_async_copy(ones_v, cnts_sh.at[lidx], scat_sem).wait()
    plsc.subcore_barrier()

    pltpu.sync_copy(sums_sh.at[pl.ds(nbase, NODES_PT)], sv)
    pltpu.sync_copy(cnts_sh.at[pl.ds(nbase, NODES_PT)], cv)

    def obody(i, _):
        sl = pl.ds(i * L, L)
        ov[sl] = sv[sl] / jnp.maximum(cv[sl], 1.0)
        return 0
    lax.fori_loop(0, NODES_PT // L, obody, 0)

    @pl.when(t < NS - 1)
    def _():
        pltpu.sync_copy(ov, out_hbm.at[pl.ds(nbase, NODES_PT)])

    @pl.when(t == NS - 1)
    def _():
        pltpu.sync_copy(ov.at[pl.ds(0, LAST_N)],
                        out_hbm.at[pl.ds(nbase, LAST_N)])


@jax.jit
def kernel(x, edge_index, edge_attr):
    xcol = jnp.pad(x[:, 0], (0, N_PAD - N_NODES))
    dst_p = jnp.pad(edge_index[1].reshape(NS, EPE),
                    ((0, 0), (0, EPT - EPE)),
                    constant_values=N_PAD - 1).reshape(NS, CPT, CH)
    return _sc_kernel(xcol, edge_index.reshape(-1), edge_attr[:, 0], dst_p)


# speedup vs baseline: 1.2527x; 1.2527x over previous
import functools

import jax
import jax.numpy as jnp
from jax import lax
from jax.experimental import pallas as pl
from jax.experimental.pallas import tpu as pltpu
from jax.experimental.pallas import tpu_sc as plsc

N_NODES = 10000
N_EDGES = 320000
NS = 16
L = 16
CH = 128
VPC = CH // L
N_PAD = 10240
NODES_PT = N_PAD // NS
LAST_N = N_NODES - (NS - 1) * NODES_PT
CPT = -(-N_EDGES // (NS * CH))
EPT = CPT * CH
EPE = N_EDGES // NS
E_PAD = EPT * NS

_mesh = plsc.VectorSubcoreMesh(
    core_axis_name="c", subcore_axis_name="s", num_cores=1, num_subcores=NS)


@functools.partial(
    pl.kernel,
    out_type=jax.ShapeDtypeStruct((N_NODES,), jnp.float32),
    mesh=_mesh,
    compiler_params=pltpu.CompilerParams(needs_layout_passes=False),
    scratch_types=[
        pltpu.VMEM((N_PAD,), jnp.float32),
        pltpu.VMEM((EPT,), jnp.int32),
        pltpu.VMEM((EPT,), jnp.int32),
        pltpu.VMEM((EPT,), jnp.float32),
        pltpu.VMEM((CPT, CH), jnp.float32),
        pltpu.VMEM((CH,), jnp.float32),
        pltpu.VMEM((NODES_PT,), jnp.float32),
        pltpu.VMEM((NODES_PT,), jnp.float32),
        pltpu.VMEM((NODES_PT,), jnp.float32),
        pltpu.VMEM_SHARED((N_PAD,), jnp.float32),
        pltpu.VMEM_SHARED((N_PAD,), jnp.float32),
        pltpu.SemaphoreType.DMA,
        pltpu.SemaphoreType.DMA,
    ],
)
def _sc_kernel(xcol_hbm, ei_hbm, ea_hbm, out_hbm,
               xcol_v, src_v, dst_v, ea_v, ld_v, ones_v,
               sv, cv, ov, sums_sh, cnts_sh, stage_sem, scat_sem):
    t = lax.axis_index("s")
    nbase = t * NODES_PT
    ebase = t * EPE

    c0 = pltpu.async_copy(xcol_hbm, xcol_v, stage_sem)
    c1 = pltpu.async_copy(ei_hbm.at[pl.ds(ebase, EPE)],
                          src_v.at[pl.ds(0, EPE)], stage_sem)
    c2 = pltpu.async_copy(ei_hbm.at[pl.ds(N_EDGES + ebase, EPE)],
                          dst_v.at[pl.ds(0, EPE)], stage_sem)
    c3 = pltpu.async_copy(ea_hbm.at[pl.ds(ebase, EPE)],
                          ea_v.at[pl.ds(0, EPE)], stage_sem)

    for i in range((EPT - EPE) // L):
        src_v[pl.ds(EPE + i * L, L)] = jnp.zeros((L,), jnp.int32)
        dst_v[pl.ds(EPE + i * L, L)] = jnp.full((L,), N_PAD - 1, jnp.int32)
        ea_v[pl.ds(EPE + i * L, L)] = jnp.ones((L,), jnp.float32)

    def zbody(i, _):
        ov[pl.ds(i * L, L)] = jnp.zeros((L,), jnp.float32)
        return 0
    lax.fori_loop(0, NODES_PT // L, zbody, 0)
    for i in range(VPC):
        ones_v[pl.ds(i * L, L)] = jnp.ones((L,), jnp.float32)
    pltpu.sync_copy(ov, sums_sh.at[pl.ds(nbase, NODES_PT)])
    pltpu.sync_copy(ov, cnts_sh.at[pl.ds(nbase, NODES_PT)])
    c0.wait()
    c1.wait()
    c2.wait()
    c3.wait()
    plsc.subcore_barrier()

    @plsc.parallel_loop(0, CPT)
    def _compute(j):
        for k in range(VPC):
            o = j * CH + k * L
            sl = pl.ds(k * L, L)
            xs = plsc.load_gather(xcol_v, [src_v[pl.ds(o, L)]])
            xd = plsc.load_gather(xcol_v, [dst_v[pl.ds(o, L)]])
            ld_v[j, sl] = (xd - xs) / ea_v[pl.ds(o, L)]

    def sbody(j, _):
        idx = dst_v.at[pl.ds(j * CH, CH)]
        pltpu.async_copy(ld_v.at[j], sums_sh.at[idx], scat_sem, add=True)
        pltpu.async_copy(ones_v, cnts_sh.at[idx], scat_sem, add=True)

        @pl.when(j > 0)
        def _():
            pidx = dst_v.at[pl.ds((j - 1) * CH, CH)]
            pltpu.make_async_copy(ld_v.at[j - 1], sums_sh.at[pidx],
                                  scat_sem).wait()
            pltpu.make_async_copy(ones_v, cnts_sh.at[pidx], scat_sem).wait()
        return 0
    lax.fori_loop(0, CPT, sbody, 0)
    lidx = dst_v.at[pl.ds((CPT - 1) * CH, CH)]
    pltpu.make_async_copy(ld_v.at[CPT - 1], sums_sh.at[lidx], scat_sem).wait()
    pltpu.make_async_copy(ones_v, cnts_sh.at[lidx], scat_sem).wait()
    plsc.subcore_barrier()

    pltpu.sync_copy(sums_sh.at[pl.ds(nbase, NODES_PT)], sv)
    pltpu.sync_copy(cnts_sh.at[pl.ds(nbase, NODES_PT)], cv)

    def obody(i, _):
        sl = pl.ds(i * L, L)
        ov[sl] = sv[sl] / jnp.maximum(cv[sl], 1.0)
        return 0
    lax.fori_loop(0, NODES_PT // L, obody, 0)

    @pl.when(t < NS - 1)
    def _():
        pltpu.sync_copy(ov, out_hbm.at[pl.ds(nbase, NODES_PT)])

    @pl.when(t == NS - 1)
    def _():
        pltpu.sync_copy(ov.at[pl.ds(0, LAST_N)],
                        out_hbm.at[pl.ds(nbase, LAST_N)])


@jax.jit
def kernel(x, edge_index, edge_attr):
    xcol = jnp.pad(x[:, 0], (0, N_PAD - N_NODES))
    return _sc_kernel(xcol, edge_index.reshape(-1), edge_attr[:, 0])
